# baseline (speedup 1.0000x reference)
import jax
import jax.numpy as jnp
from jax.experimental import pallas as pl
from jax.experimental.pallas import tpu as pltpu

HID = 32
STACK = 4
TILE_B = 4096


def _round_up(x, m):
    return (x + m - 1) // m * m


def _fused_mlp_kernel(x_ref, w1_ref, b1_ref, w2bd_ref, b2s_ref, w3bd_ref,
                      b3s_ref, w4s_ref, b4_ref, o_ref):
    t4 = x_ref.shape[1] // STACK

    chunks = []
    for c in range(STACK):
        xc = x_ref[:, c * t4:(c + 1) * t4]
        h = (b1_ref[...]
             + w1_ref[:, 0:1] * xc[0:1, :]
             + w1_ref[:, 1:2] * xc[1:2, :]
             + w1_ref[:, 2:3] * xc[2:3, :])
        chunks.append(h)
    h1 = jnp.maximum(jnp.concatenate(chunks, axis=0), 0.0)

    h2 = jnp.dot(w2bd_ref[...], h1.astype(jnp.bfloat16),
                 preferred_element_type=jnp.float32) + b2s_ref[...]
    h2 = jnp.maximum(h2, 0.0)
    h3 = jnp.dot(w3bd_ref[...], h2.astype(jnp.bfloat16),
                 preferred_element_type=jnp.float32) + b3s_ref[...]
    h3 = jnp.maximum(h3, 0.0)

    o4 = jnp.dot(w4s_ref[...], h3.astype(jnp.bfloat16),
                 preferred_element_type=jnp.float32)
    for c in range(STACK):
        o_ref[0:1, c * t4:(c + 1) * t4] = o4[c:c + 1, :] + b4_ref[...]


@jax.jit
def _forward(x, w1, b1, w2, b2, w3, b3, w4, b4):
    n = x.shape[0]
    n_pad = _round_up(max(n, 1), TILE_B)

    x_t = jnp.zeros((3, n_pad), jnp.float32).at[:, :n].set(x.T)

    eye = jnp.eye(STACK, dtype=jnp.float32)
    w2bd = jnp.kron(eye, w2).astype(jnp.bfloat16)
    w3bd = jnp.kron(eye, w3).astype(jnp.bfloat16)
    b2s = jnp.tile(b2, (STACK, 1))
    b3s = jnp.tile(b3, (STACK, 1))
    w4s = jnp.zeros((8, STACK * HID), jnp.float32).at[:STACK].set(
        jnp.kron(eye, w4[0:1, :])).astype(jnp.bfloat16)

    grid = (n_pad // TILE_B,)
    out_t = pl.pallas_call(
        _fused_mlp_kernel,
        out_shape=jax.ShapeDtypeStruct((1, n_pad), jnp.float32),
        grid=grid,
        in_specs=[
            pl.BlockSpec((3, TILE_B), lambda i: (0, i)),
            pl.BlockSpec((HID, 3), lambda i: (0, 0)),
            pl.BlockSpec((HID, 1), lambda i: (0, 0)),
            pl.BlockSpec((128, 128), lambda i: (0, 0)),
            pl.BlockSpec((128, 1), lambda i: (0, 0)),
            pl.BlockSpec((128, 128), lambda i: (0, 0)),
            pl.BlockSpec((128, 1), lambda i: (0, 0)),
            pl.BlockSpec((8, 128), lambda i: (0, 0)),
            pl.BlockSpec((1, 1), lambda i: (0, 0)),
        ],
        out_specs=pl.BlockSpec((1, TILE_B), lambda i: (0, i)),
        compiler_params=pltpu.CompilerParams(
            dimension_semantics=("parallel",),
            vmem_limit_bytes=64 * 1024 * 1024,
        ),
    )(x_t, w1, b1, w2bd, b2s, w3bd, b3s, w4s, b4)

    return out_t[:1, :n].T


def kernel(x, w1, b1, w2, b2, w3, b3, w4, b4):
    return _forward(x, w1, b1, w2, b2, w3, b3, w4, b4)
